# Initial kernel scaffold; baseline (speedup 1.0000x reference)
import functools

import jax
import jax.numpy as jnp
from jax import lax
from jax.experimental import pallas as pl
from jax.experimental.pallas import tpu as pltpu
from jax.experimental.pallas import tpu_sc as plsc

NC = 2
NS = 16
NW = NC * NS
CH = 128
LANES = 16


def kernel(input_ids, token_table, position_table):
    B, S = input_ids.shape
    V, D = token_table.shape
    N = B * S
    assert N % (NW * CH) == 0
    CPW = N // (NW * CH)

    ids2d = input_ids.reshape(N // CH, CH).astype(jnp.int32)

    mesh = plsc.VectorSubcoreMesh(core_axis_name="c", subcore_axis_name="s")

    @functools.partial(
        pl.kernel,
        out_type=jax.ShapeDtypeStruct((N, D), jnp.float32),
        mesh=mesh,
        scratch_types=[
            pltpu.VMEM((CPW, CH), jnp.int32),
            pltpu.VMEM((S, D), jnp.float32),
            pltpu.VMEM((CH, D), jnp.float32),
            pltpu.SemaphoreType.DMA,
        ],
    )
    def run(ids_hbm, tok_hbm, pos_hbm, out_hbm, idx_v, pos_v, rows_v, sem):
        wid = lax.axis_index("s") * NC + lax.axis_index("c")
        row0 = wid * CPW * CH
        pltpu.sync_copy(ids_hbm.at[pl.ds(wid * CPW, CPW)], idx_v)
        pltpu.sync_copy(pos_hbm, pos_v)

        def chunk_body(j, carry):
            pltpu.async_copy(tok_hbm.at[idx_v.at[j]], rows_v, sem).wait()

            def row_body(r, carry2):
                p = lax.rem(row0 + j * CH + r, S)
                for c in range(D // LANES):
                    pv = pos_v[p, pl.ds(c * LANES, LANES)]
                    plsc.addupdate(rows_v.at[r, pl.ds(c * LANES, LANES)], pv)
                return carry2

            lax.fori_loop(0, CH, row_body, 0)
            pltpu.sync_copy(rows_v, out_hbm.at[pl.ds(row0 + j * CH, CH)])
            return carry

        lax.fori_loop(0, CPW, chunk_body, 0)

    out = run(ids2d, token_table, position_table)
    return out.reshape(B, S, D)


# speedup vs baseline: 2.0354x; 2.0354x over previous
import functools

import jax
import jax.numpy as jnp
from jax import lax
from jax.experimental import pallas as pl
from jax.experimental.pallas import tpu as pltpu
from jax.experimental.pallas import tpu_sc as plsc

NC = 2
NS = 16
NW = NC * NS
CH = 128
LANES = 16


def kernel(input_ids, token_table, position_table):
    B, S = input_ids.shape
    V, D = token_table.shape
    N = B * S
    assert N % (NW * CH) == 0
    CPW = N // (NW * CH)

    ids2d = input_ids.reshape(N // CH, CH).astype(jnp.int32)

    mesh = plsc.VectorSubcoreMesh(
        core_axis_name="c", subcore_axis_name="s", num_cores=NC, num_subcores=NS
    )

    @functools.partial(
        pl.kernel,
        out_type=jax.ShapeDtypeStruct((N, D), jnp.float32),
        mesh=mesh,
        scratch_types=[
            pltpu.VMEM((CPW, CH), jnp.int32),
            pltpu.VMEM((S, D), jnp.float32),
            pltpu.VMEM((CH, D), jnp.float32),
            pltpu.SemaphoreType.DMA,
        ],
        compiler_params=pltpu.CompilerParams(use_tc_tiling_on_sc=False),
    )
    def run(ids_hbm, tok_hbm, pos_hbm, out_hbm, idx_v, pos_v, rows_v, sem):
        wid = lax.axis_index("s") * NC + lax.axis_index("c")
        row0 = wid * CPW * CH
        pltpu.sync_copy(ids_hbm.at[pl.ds(wid * CPW, CPW)], idx_v)
        pltpu.sync_copy(pos_hbm, pos_v)

        def chunk_body(j, carry):
            pltpu.async_copy(tok_hbm.at[idx_v.at[j]], rows_v, sem).wait()

            def row_body(r, carry2):
                p = lax.rem(row0 + j * CH + r, S)
                for c in range(D // LANES):
                    pv = pos_v[p, pl.ds(c * LANES, LANES)]
                    plsc.addupdate(rows_v.at[r, pl.ds(c * LANES, LANES)], pv)
                return carry2

            lax.fori_loop(0, CH, row_body, 0)
            pltpu.sync_copy(rows_v, out_hbm.at[pl.ds(row0 + j * CH, CH)])
            return carry

        lax.fori_loop(0, CPW, chunk_body, 0)

    out = run(ids2d, token_table, position_table)
    return out.reshape(B, S, D)


# speedup vs baseline: 2.4570x; 1.2072x over previous
import functools

import jax
import jax.numpy as jnp
from jax import lax
from jax.experimental import pallas as pl
from jax.experimental.pallas import tpu as pltpu
from jax.experimental.pallas import tpu_sc as plsc

NC = 2
NS = 16
NW = NC * NS
CH = 128
LANES = 16


def kernel(input_ids, token_table, position_table):
    B, S = input_ids.shape
    V, D = token_table.shape
    N = B * S
    assert N % (NW * CH) == 0
    CPW = N // (NW * CH)

    ids2d = input_ids.reshape(N // CH, CH).astype(jnp.int32)

    pos2 = jnp.concatenate([position_table, position_table], axis=0)

    mesh = plsc.VectorSubcoreMesh(
        core_axis_name="c", subcore_axis_name="s", num_cores=NC, num_subcores=NS
    )

    @functools.partial(
        pl.kernel,
        out_type=jax.ShapeDtypeStruct((N, D), jnp.float32),
        mesh=mesh,
        scratch_types=[
            pltpu.VMEM((CPW, CH), jnp.int32),
            pltpu.VMEM_SHARED((2 * S, D), jnp.float32),
            pltpu.VMEM((CH, D), jnp.float32),
            pltpu.VMEM((CH,), jnp.int32),
            pltpu.SemaphoreType.DMA,
        ],
        compiler_params=pltpu.CompilerParams(use_tc_tiling_on_sc=False),
    )
    def run(ids_hbm, tok_hbm, pos_hbm, out_hbm, idx_v, pos_v, rows_v, iden_v, sem):
        del iden_v
        wid = lax.axis_index("s") * NC + lax.axis_index("c")
        row0 = wid * CPW * CH
        pltpu.sync_copy(ids_hbm.at[pl.ds(wid * CPW, CPW)], idx_v)
        pltpu.sync_copy(pos_hbm, pos_v)

        def chunk_body(j, carry):
            p0 = lax.rem(row0 + j * CH, S)
            pltpu.sync_copy(pos_v.at[pl.ds(p0, CH)], rows_v)
            pltpu.async_copy(tok_hbm.at[idx_v.at[j]], rows_v, sem, add=True).wait()
            pltpu.sync_copy(rows_v, out_hbm.at[pl.ds(row0 + j * CH, CH)])
            return carry

        lax.fori_loop(0, CPW, chunk_body, 0)

    out = run(ids2d, token_table, pos2)
    return out.reshape(B, S, D)


# speedup vs baseline: 2.8659x; 1.1664x over previous
import functools

import jax
import jax.numpy as jnp
from jax import lax
from jax.experimental import pallas as pl
from jax.experimental.pallas import tpu as pltpu
from jax.experimental.pallas import tpu_sc as plsc

NC = 2
NS = 16
NW = NC * NS
CH = 128
NBUF = 8
LA = 2


def kernel(input_ids, token_table, position_table):
    B, S = input_ids.shape
    V, D = token_table.shape
    N = B * S
    assert N % (NW * CH) == 0
    CPW = N // (NW * CH)
    assert CPW % NBUF == 0 and CH <= S

    ids2d = input_ids.reshape(N // CH, CH).astype(jnp.int32)

    pos2 = jnp.concatenate([position_table, position_table], axis=0)

    mesh = plsc.VectorSubcoreMesh(
        core_axis_name="c", subcore_axis_name="s", num_cores=NC, num_subcores=NS
    )

    @functools.partial(
        pl.kernel,
        out_type=jax.ShapeDtypeStruct((N, D), jnp.float32),
        mesh=mesh,
        scratch_types=[
            pltpu.VMEM((CPW, CH), jnp.int32),
            pltpu.VMEM_SHARED((2 * S, D), jnp.float32),
        ]
        + [pltpu.VMEM((CH, D), jnp.float32)] * NBUF
        + [pltpu.SemaphoreType.DMA] * (3 * NBUF),
        compiler_params=pltpu.CompilerParams(use_tc_tiling_on_sc=False),
    )
    def run(ids_hbm, tok_hbm, pos_hbm, out_hbm, idx_v, pos_sp, *rest):
        rows = rest[:NBUF]
        psem = rest[NBUF : 2 * NBUF]
        gsem = rest[2 * NBUF : 3 * NBUF]
        wsem = rest[3 * NBUF : 4 * NBUF]

        sid = lax.axis_index("s")
        wid = sid * NC + lax.axis_index("c")
        row0 = wid * CPW * CH
        T = CPW

        @pl.when(sid == 0)
        def _():
            pltpu.sync_copy(pos_hbm, pos_sp)

        pltpu.sync_copy(ids_hbm.at[pl.ds(wid * CPW, CPW)], idx_v)
        plsc.subcore_barrier()

        def start_prefill(j, b):
            p0 = lax.rem(row0 + j * CH, S)
            pltpu.async_copy(pos_sp.at[pl.ds(p0, CH)], rows[b], psem[b])

        def wait_prefill(b):
            pltpu.make_async_copy(pos_sp.at[pl.ds(0, CH)], rows[b], psem[b]).wait()

        def start_gather(j, b):
            pltpu.async_copy(tok_hbm.at[idx_v.at[j]], rows[b], gsem[b], add=True)

        def wait_gather(j, b):
            pltpu.make_async_copy(tok_hbm.at[idx_v.at[j]], rows[b], gsem[b]).wait()

        def start_write(j, b):
            pltpu.async_copy(rows[b], out_hbm.at[pl.ds(row0 + j * CH, CH)], wsem[b])

        def wait_write(b):
            pltpu.make_async_copy(rows[b], out_hbm.at[pl.ds(0, CH)], wsem[b]).wait()

        for jj in range(LA + 1):
            start_prefill(jj, jj)
        for jj in range(LA):
            wait_prefill(jj)
            start_gather(jj, jj)

        def outer(g, carry):
            jb = g * NBUF
            for b in range(NBUF):
                j = jb + b
                bP = (b + LA + 1) % NBUF
                bG = (b + LA) % NBUF
                jP = j + LA + 1
                jG = j + LA

                @pl.when(jP < T)
                def _():
                    @pl.when(jP >= NBUF)
                    def _():
                        wait_write(bP)

                    start_prefill(jP, bP)

                @pl.when(jG < T)
                def _():
                    wait_prefill(bG)
                    start_gather(jG, bG)

                wait_gather(j, b)
                start_write(j, b)
            return carry

        lax.fori_loop(0, T // NBUF, outer, 0)
        for b in range(NBUF):
            wait_write(b)

    out = run(ids2d, token_table, pos2)
    return out.reshape(B, S, D)


# speedup vs baseline: 2.8704x; 1.0016x over previous
import functools

import jax
import jax.numpy as jnp
from jax import lax
from jax.experimental import pallas as pl
from jax.experimental.pallas import tpu as pltpu
from jax.experimental.pallas import tpu_sc as plsc

NC = 2
NS = 16
NW = NC * NS
CH = 128
NBUF = 8
LA = 4


def kernel(input_ids, token_table, position_table):
    B, S = input_ids.shape
    V, D = token_table.shape
    N = B * S
    assert N % (NW * CH) == 0
    CPW = N // (NW * CH)
    assert CPW % NBUF == 0 and CH <= S

    ids2d = input_ids.reshape(N // CH, CH).astype(jnp.int32)

    pos2 = jnp.concatenate([position_table, position_table], axis=0)

    mesh = plsc.VectorSubcoreMesh(
        core_axis_name="c", subcore_axis_name="s", num_cores=NC, num_subcores=NS
    )

    @functools.partial(
        pl.kernel,
        out_type=jax.ShapeDtypeStruct((N, D), jnp.float32),
        mesh=mesh,
        scratch_types=[
            pltpu.VMEM((CPW, CH), jnp.int32),
            pltpu.VMEM_SHARED((2 * S, D), jnp.float32),
        ]
        + [pltpu.VMEM((CH, D), jnp.float32)] * NBUF
        + [pltpu.SemaphoreType.DMA] * (3 * NBUF),
        compiler_params=pltpu.CompilerParams(use_tc_tiling_on_sc=False),
    )
    def run(ids_hbm, tok_hbm, pos_hbm, out_hbm, idx_v, pos_sp, *rest):
        rows = rest[:NBUF]
        psem = rest[NBUF : 2 * NBUF]
        gsem = rest[2 * NBUF : 3 * NBUF]
        wsem = rest[3 * NBUF : 4 * NBUF]

        sid = lax.axis_index("s")
        wid = sid * NC + lax.axis_index("c")
        row0 = wid * CPW * CH
        T = CPW

        @pl.when(sid == 0)
        def _():
            pltpu.sync_copy(pos_hbm, pos_sp)

        pltpu.sync_copy(ids_hbm.at[pl.ds(wid * CPW, CPW)], idx_v)
        plsc.subcore_barrier()

        def start_prefill(j, b):
            p0 = lax.rem(row0 + j * CH, S)
            pltpu.async_copy(pos_sp.at[pl.ds(p0, CH)], rows[b], psem[b])

        def wait_prefill(b):
            pltpu.make_async_copy(pos_sp.at[pl.ds(0, CH)], rows[b], psem[b]).wait()

        def start_gather(j, b):
            pltpu.async_copy(tok_hbm.at[idx_v.at[j]], rows[b], gsem[b], add=True)

        def wait_gather(j, b):
            pltpu.make_async_copy(tok_hbm.at[idx_v.at[j]], rows[b], gsem[b]).wait()

        def start_write(j, b):
            pltpu.async_copy(rows[b], out_hbm.at[pl.ds(row0 + j * CH, CH)], wsem[b])

        def wait_write(b):
            pltpu.make_async_copy(rows[b], out_hbm.at[pl.ds(0, CH)], wsem[b]).wait()

        for jj in range(LA + 1):
            start_prefill(jj, jj)
        for jj in range(LA):
            wait_prefill(jj)
            start_gather(jj, jj)

        def outer(g, carry):
            jb = g * NBUF
            for b in range(NBUF):
                j = jb + b
                bP = (b + LA + 1) % NBUF
                bG = (b + LA) % NBUF
                jP = j + LA + 1
                jG = j + LA

                @pl.when(jP < T)
                def _():
                    @pl.when(jP >= NBUF)
                    def _():
                        wait_write(bP)

                    start_prefill(jP, bP)

                @pl.when(jG < T)
                def _():
                    wait_prefill(bG)
                    start_gather(jG, bG)

                wait_gather(j, b)
                start_write(j, b)
            return carry

        lax.fori_loop(0, T // NBUF, outer, 0)
        for b in range(NBUF):
            wait_write(b)

    out = run(ids2d, token_table, pos2)
    return out.reshape(B, S, D)
